# Initial kernel scaffold; baseline (speedup 1.0000x reference)
import functools

import jax
import jax.numpy as jnp
from jax import lax
from jax.experimental import pallas as pl
from jax.experimental.pallas import tpu as pltpu

_N = 16384
_D = 256
_K = 512
_ITERS = 10
_BN = 2048
_NB = _N // _BN


def _lloyd_body(data_ref, initT_ref, centsT_out, labels_ref, loss_ref,
                centsT_s, sumsT_s, counts_s):
    i = pl.program_id(0)
    j = pl.program_id(1)

    @pl.when((i == 0) & (j == 0))
    def _():
        centsT_s[...] = initT_ref[...]

    @pl.when(j == 0)
    def _():
        sumsT_s[...] = jnp.zeros((_D, _K), jnp.float32)
        counts_s[...] = jnp.zeros((_K,), jnp.float32)

    centsT = centsT_s[...]
    x_blk = data_ref[...]

    c2 = jnp.sum(centsT * centsT, axis=0)
    x2 = jnp.sum(x_blk * x_blk, axis=1)
    prod = lax.dot_general(x_blk, centsT, (((1,), (0,)), ((), ())),
                           preferred_element_type=jnp.float32)
    d2 = x2[:, None] - 2.0 * prod + c2[None, :]
    d2 = jnp.maximum(d2, 0.0)
    labels = jnp.argmin(d2, axis=1).astype(jnp.int32)
    labels_ref[...] = labels

    @pl.when(i < _ITERS)
    def _():
        iota_k = lax.broadcasted_iota(jnp.int32, (1, _K), 1)
        onehot = (labels[:, None] == iota_k).astype(jnp.float32)
        psums = lax.dot_general(x_blk, onehot, (((0,), (0,)), ((), ())),
                                preferred_element_type=jnp.float32)
        sumsT_s[...] += psums
        counts_s[...] += jnp.sum(onehot, axis=0)

    @pl.when((i < _ITERS) & (j == _NB - 1))
    def _():
        counts = counts_s[...]
        sumsT = sumsT_s[...]
        newT = jnp.where(counts[None, :] > 0,
                         sumsT / jnp.maximum(counts, 1.0)[None, :],
                         centsT)
        centsT_s[...] = newT

    @pl.when(i == _ITERS)
    def _():
        mind = jnp.min(d2, axis=1)
        blk = jnp.sum(mind)

        @pl.when(j == 0)
        def _():
            loss_ref[0, 0] = blk

        @pl.when(j > 0)
        def _():
            loss_ref[0, 0] += blk

        @pl.when(j == _NB - 1)
        def _():
            centsT_out[...] = centsT_s[...]


@functools.partial(jax.jit, static_argnames=("interpret",))
def _lloyd(data, initT, interpret=False):
    centsT, labels, loss = pl.pallas_call(
        _lloyd_body,
        grid=(_ITERS + 1, _NB),
        in_specs=[
            pl.BlockSpec((_BN, _D), lambda i, j: (j, 0)),
            pl.BlockSpec((_D, _K), lambda i, j: (0, 0)),
        ],
        out_specs=[
            pl.BlockSpec((_D, _K), lambda i, j: (0, 0)),
            pl.BlockSpec((_BN,), lambda i, j: (j,)),
            pl.BlockSpec((1, 1), lambda i, j: (0, 0)),
        ],
        out_shape=[
            jax.ShapeDtypeStruct((_D, _K), jnp.float32),
            jax.ShapeDtypeStruct((_N,), jnp.int32),
            jax.ShapeDtypeStruct((1, 1), jnp.float32),
        ],
        scratch_shapes=[
            pltpu.VMEM((_D, _K), jnp.float32),
            pltpu.VMEM((_D, _K), jnp.float32),
            pltpu.VMEM((_K,), jnp.float32),
        ],
        compiler_params=pltpu.CompilerParams(
            dimension_semantics=("arbitrary", "arbitrary"),
        ),
        interpret=interpret,
    )(data, initT)
    return centsT.T, labels, loss[0, 0]


def kernel(data, seed):
    key = jax.random.key(seed)
    idx = jax.random.choice(key, data.shape[0], (_K,), replace=False)
    initT = data[idx].T
    return _lloyd(data, initT)


# speedup vs baseline: 2.4458x; 2.4458x over previous
import functools

import jax
import jax.numpy as jnp
from jax import lax
from jax.experimental import pallas as pl
from jax.experimental.pallas import tpu as pltpu

_N = 16384
_D = 256
_K = 512
_ITERS = 10
_BN = 2048
_NB = _N // _BN


def _iter_body(data_ref, cents_ref, x2_ref, c2_ref, newc_ref,
               sums_s, counts_s):
    j = pl.program_id(0)

    @pl.when(j == 0)
    def _():
        sums_s[...] = jnp.zeros((_K, _D), jnp.float32)
        counts_s[...] = jnp.zeros((_K,), jnp.float32)

    cents = cents_ref[...]
    x_blk = data_ref[...]
    prod = lax.dot_general(x_blk, cents, (((1,), (1,)), ((), ())),
                           preferred_element_type=jnp.float32)
    d2 = x2_ref[...][:, None] - 2.0 * prod + c2_ref[...][None, :]
    d2 = jnp.maximum(d2, 0.0)
    labels = jnp.argmin(d2, axis=1).astype(jnp.int32)

    iota_k = lax.broadcasted_iota(jnp.int32, (1, _K), 1)
    onehot = (labels[:, None] == iota_k).astype(jnp.float32)
    psums = lax.dot_general(onehot, x_blk, (((0,), (0,)), ((), ())),
                            preferred_element_type=jnp.float32,
                            precision=lax.Precision.HIGHEST)
    sums_s[...] += psums
    counts_s[...] += jnp.sum(onehot, axis=0)

    @pl.when(j == _NB - 1)
    def _():
        counts = counts_s[...]
        sums = sums_s[...]
        newc = jnp.where(counts[:, None] > 0,
                         sums / jnp.maximum(counts, 1.0)[:, None],
                         cents)
        newc_ref[...] = newc


def _final_body(data_ref, cents_ref, x2_ref, c2_ref, labels_ref, loss_ref):
    j = pl.program_id(0)
    cents = cents_ref[...]
    x_blk = data_ref[...]
    prod = lax.dot_general(x_blk, cents, (((1,), (1,)), ((), ())),
                           preferred_element_type=jnp.float32)
    d2 = x2_ref[...][:, None] - 2.0 * prod + c2_ref[...][None, :]
    d2 = jnp.maximum(d2, 0.0)
    labels_ref[...] = jnp.argmin(d2, axis=1).astype(jnp.int32)
    blk = jnp.reshape(jnp.sum(jnp.min(d2, axis=1)), (1, 1))

    @pl.when(j == 0)
    def _():
        loss_ref[...] = blk

    @pl.when(j > 0)
    def _():
        loss_ref[...] += blk


_iter_call = pl.pallas_call(
    _iter_body,
    grid=(_NB,),
    in_specs=[
        pl.BlockSpec((_BN, _D), lambda j: (j, 0)),
        pl.BlockSpec((_K, _D), lambda j: (0, 0)),
        pl.BlockSpec((_BN,), lambda j: (j,)),
        pl.BlockSpec((_K,), lambda j: (0,)),
    ],
    out_specs=pl.BlockSpec((_K, _D), lambda j: (0, 0)),
    out_shape=jax.ShapeDtypeStruct((_K, _D), jnp.float32),
    scratch_shapes=[
        pltpu.VMEM((_K, _D), jnp.float32),
        pltpu.VMEM((_K,), jnp.float32),
    ],
    compiler_params=pltpu.CompilerParams(
        dimension_semantics=("arbitrary",),
    ),
)

_final_call = pl.pallas_call(
    _final_body,
    grid=(_NB,),
    in_specs=[
        pl.BlockSpec((_BN, _D), lambda j: (j, 0)),
        pl.BlockSpec((_K, _D), lambda j: (0, 0)),
        pl.BlockSpec((_BN,), lambda j: (j,)),
        pl.BlockSpec((_K,), lambda j: (0,)),
    ],
    out_specs=[
        pl.BlockSpec((_BN,), lambda j: (j,)),
        pl.BlockSpec((1, 1), lambda j: (0, 0)),
    ],
    out_shape=[
        jax.ShapeDtypeStruct((_N,), jnp.int32),
        jax.ShapeDtypeStruct((1, 1), jnp.float32),
    ],
    compiler_params=pltpu.CompilerParams(
        dimension_semantics=("arbitrary",),
    ),
)


@jax.jit
def _lloyd(data, init_cents):
    x2 = jnp.sum(data * data, axis=1)

    def step(cents, _):
        c2 = jnp.sum(cents * cents, axis=1)
        new_cents = _iter_call(data, cents, x2, c2)
        return new_cents, None

    cents, _ = lax.scan(step, init_cents, None, length=_ITERS)
    c2 = jnp.sum(cents * cents, axis=1)
    labels, loss = _final_call(data, cents, x2, c2)
    return cents, labels, loss[0, 0]


def kernel(data, seed):
    key = jax.random.key(seed)
    idx = jax.random.choice(key, data.shape[0], (_K,), replace=False)
    return _lloyd(data, data[idx])


# speedup vs baseline: 2.4947x; 1.0200x over previous
import functools

import jax
import jax.numpy as jnp
from jax import lax
from jax.experimental import pallas as pl
from jax.experimental.pallas import tpu as pltpu

_N = 16384
_D = 256
_K = 512
_ITERS = 10
_BN = 4096
_NB = _N // _BN


def _iter_body(data_ref, cents_ref, x2_ref, c2_ref, newc_ref,
               sums_s, counts_s):
    j = pl.program_id(0)

    @pl.when(j == 0)
    def _():
        sums_s[...] = jnp.zeros((_K, _D), jnp.float32)
        counts_s[...] = jnp.zeros((_K,), jnp.float32)

    cents = cents_ref[...]
    x_blk = data_ref[...]
    prod = lax.dot_general(x_blk, cents, (((1,), (1,)), ((), ())),
                           preferred_element_type=jnp.float32)
    d2 = x2_ref[...][:, None] - 2.0 * prod + c2_ref[...][None, :]
    d2 = jnp.maximum(d2, 0.0)
    labels = jnp.argmin(d2, axis=1).astype(jnp.int32)

    iota_k = lax.broadcasted_iota(jnp.int32, (1, _K), 1)
    onehot = (labels[:, None] == iota_k).astype(jnp.float32)
    psums = lax.dot_general(onehot, x_blk, (((0,), (0,)), ((), ())),
                            preferred_element_type=jnp.float32,
                            precision=lax.Precision.HIGHEST)
    sums_s[...] += psums
    counts_s[...] += jnp.sum(onehot, axis=0)

    @pl.when(j == _NB - 1)
    def _():
        counts = counts_s[...]
        sums = sums_s[...]
        newc = jnp.where(counts[:, None] > 0,
                         sums / jnp.maximum(counts, 1.0)[:, None],
                         cents)
        newc_ref[...] = newc


def _final_body(data_ref, cents_ref, x2_ref, c2_ref, labels_ref, loss_ref):
    j = pl.program_id(0)
    cents = cents_ref[...]
    x_blk = data_ref[...]
    prod = lax.dot_general(x_blk, cents, (((1,), (1,)), ((), ())),
                           preferred_element_type=jnp.float32)
    d2 = x2_ref[...][:, None] - 2.0 * prod + c2_ref[...][None, :]
    d2 = jnp.maximum(d2, 0.0)
    labels_ref[...] = jnp.argmin(d2, axis=1).astype(jnp.int32)
    blk = jnp.reshape(jnp.sum(jnp.min(d2, axis=1)), (1, 1))

    @pl.when(j == 0)
    def _():
        loss_ref[...] = blk

    @pl.when(j > 0)
    def _():
        loss_ref[...] += blk


_iter_call = pl.pallas_call(
    _iter_body,
    grid=(_NB,),
    in_specs=[
        pl.BlockSpec((_BN, _D), lambda j: (j, 0)),
        pl.BlockSpec((_K, _D), lambda j: (0, 0)),
        pl.BlockSpec((_BN,), lambda j: (j,)),
        pl.BlockSpec((_K,), lambda j: (0,)),
    ],
    out_specs=pl.BlockSpec((_K, _D), lambda j: (0, 0)),
    out_shape=jax.ShapeDtypeStruct((_K, _D), jnp.float32),
    scratch_shapes=[
        pltpu.VMEM((_K, _D), jnp.float32),
        pltpu.VMEM((_K,), jnp.float32),
    ],
    compiler_params=pltpu.CompilerParams(
        dimension_semantics=("arbitrary",),
    ),
)

_final_call = pl.pallas_call(
    _final_body,
    grid=(_NB,),
    in_specs=[
        pl.BlockSpec((_BN, _D), lambda j: (j, 0)),
        pl.BlockSpec((_K, _D), lambda j: (0, 0)),
        pl.BlockSpec((_BN,), lambda j: (j,)),
        pl.BlockSpec((_K,), lambda j: (0,)),
    ],
    out_specs=[
        pl.BlockSpec((_BN,), lambda j: (j,)),
        pl.BlockSpec((1, 1), lambda j: (0, 0)),
    ],
    out_shape=[
        jax.ShapeDtypeStruct((_N,), jnp.int32),
        jax.ShapeDtypeStruct((1, 1), jnp.float32),
    ],
    compiler_params=pltpu.CompilerParams(
        dimension_semantics=("arbitrary",),
    ),
)


@jax.jit
def _lloyd(data, init_cents):
    x2 = jnp.sum(data * data, axis=1)

    def step(cents, _):
        c2 = jnp.sum(cents * cents, axis=1)
        new_cents = _iter_call(data, cents, x2, c2)
        return new_cents, None

    cents, _ = lax.scan(step, init_cents, None, length=_ITERS)
    c2 = jnp.sum(cents * cents, axis=1)
    labels, loss = _final_call(data, cents, x2, c2)
    return cents, labels, loss[0, 0]


def kernel(data, seed):
    key = jax.random.key(seed)
    idx = jax.random.choice(key, data.shape[0], (_K,), replace=False)
    return _lloyd(data, data[idx])
